# Initial kernel scaffold; baseline (speedup 1.0000x reference)
import jax
import jax.numpy as jnp
from jax.experimental import pallas as pl


def kernel(x, edge_attr, w_gating, W_ep, b_ep, W1, b1, W2, b2, W3, b3):
    raise NotImplementedError("write your pallas kernel here")


# speedup vs baseline: 1.1654x; 1.1654x over previous
import functools

import jax
import jax.numpy as jnp
from jax import lax
from jax.experimental import pallas as pl
from jax.experimental.pallas import tpu as pltpu

N, D, OUT, E, K = 4096, 1024, 1024, 8, 2
BT = 512
NT = N // BT


def _e_of(s):
    return 3 * (s % 3) + s // 3


def _gating_body(x_ref, ea_ref, wg_ref, wep_ref, bep_ref,
                 gates_ref, idx_ref, loss_ref, acc_ref):
    t = pl.program_id(0)

    @pl.when(t == 0)
    def _():
        acc_ref[...] = jnp.zeros_like(acc_ref)

    logits = jnp.dot(x_ref[...], wg_ref[...],
                     preferred_element_type=jnp.float32)
    ea = ea_ref[...].astype(jnp.bfloat16).astype(jnp.float32)
    wep = wep_ref[...].astype(jnp.bfloat16).astype(jnp.float32)
    logits = logits + (ea[:, 0:1] * wep[0:1, :] + ea[:, 1:2] * wep[1:2, :])
    logits = logits + bep_ref[...]

    iota = lax.broadcasted_iota(jnp.int32, (BT, E), 1)
    m1 = jnp.max(logits, axis=1, keepdims=True)
    i1 = jnp.min(jnp.where(logits == m1, iota, E), axis=1, keepdims=True)
    masked = jnp.where(iota == i1, -jnp.inf, logits)
    m2 = jnp.max(masked, axis=1, keepdims=True)
    i2 = jnp.min(jnp.where(masked == m2, iota, E), axis=1, keepdims=True)
    tsm = jnp.exp(m2 - m1)
    denom = 1.0 + tsm
    g1 = 1.0 / denom
    g2 = tsm / denom

    gates_ref[...] = jnp.concatenate([g1, g2], axis=1)
    idx_ref[...] = jnp.concatenate([i1, i2], axis=1)

    oh1 = (iota == i1).astype(jnp.float32)
    oh2 = (iota == i2).astype(jnp.float32)
    acc_ref[0:1, :] += jnp.sum(g1 * oh1 + g2 * oh2, axis=0, keepdims=True)
    acc_ref[1:2, :] += jnp.sum(oh1 + oh2, axis=0, keepdims=True)

    @pl.when(t == NT - 1)
    def _():
        def cv2(v):
            mean = jnp.mean(v)
            var = jnp.sum((v - mean) ** 2) / (E - 1)
            return var / (mean * mean + 1e-10)
        loss = 0.01 * (cv2(acc_ref[0:1, :]) + cv2(acc_ref[1:2, :]))
        loss_ref[...] = jnp.broadcast_to(loss, (1, 1))


def _gating(xb, edge_attr, w_gating, W_ep, b_ep):
    return pl.pallas_call(
        _gating_body,
        grid=(NT,),
        in_specs=[
            pl.BlockSpec((BT, D), lambda t: (t, 0)),
            pl.BlockSpec((BT, 2), lambda t: (t, 0)),
            pl.BlockSpec((D, E), lambda t: (0, 0)),
            pl.BlockSpec((2, E), lambda t: (0, 0)),
            pl.BlockSpec((1, E), lambda t: (0, 0)),
        ],
        out_specs=[
            pl.BlockSpec((BT, K), lambda t: (t, 0)),
            pl.BlockSpec((BT, K), lambda t: (t, 0)),
            pl.BlockSpec((1, 1), lambda t: (0, 0)),
        ],
        out_shape=[
            jax.ShapeDtypeStruct((N, K), jnp.float32),
            jax.ShapeDtypeStruct((N, K), jnp.int32),
            jax.ShapeDtypeStruct((1, 1), jnp.float32),
        ],
        scratch_shapes=[pltpu.VMEM((2, E), jnp.float32)],
        compiler_params=pltpu.CompilerParams(
            dimension_semantics=("arbitrary",)),
    )(xb, edge_attr, w_gating.astype(jnp.bfloat16), W_ep, b_ep.reshape(1, E))


def _experts_body(xb_ref, w1_ref, b1_ref, w2_ref, b2_ref, w3_ref, b3_ref,
                  eo_ref):
    s = pl.program_id(0)
    t = pl.program_id(1)
    lyr = s // 3
    x = xb_ref[pl.ds(t * BT, BT), :]
    h1 = jnp.dot(x, w1_ref[0], preferred_element_type=jnp.float32)
    h1 = h1 + b1_ref[0]

    @pl.when(lyr == 0)
    def _():
        eo_ref[0] = h1

    @pl.when(lyr > 0)
    def _():
        h1b = jnp.maximum(h1, 0.0).astype(jnp.bfloat16)
        h2 = jnp.dot(h1b, w2_ref[0], preferred_element_type=jnp.float32)
        h2 = h2 + b2_ref[0]

        @pl.when(lyr == 1)
        def _():
            eo_ref[0] = h2

        @pl.when(lyr == 2)
        def _():
            h2b = jnp.maximum(h2, 0.0).astype(jnp.bfloat16)
            h3 = jnp.dot(h2b, w3_ref[0], preferred_element_type=jnp.float32)
            eo_ref[0] = h3 + b3_ref[0]


def _experts(xb, W1b, b1, W2b, b2, W3b, b3):
    wspec = pl.BlockSpec((1, D, OUT), lambda s, t: (_e_of(s), 0, 0))
    bspec = pl.BlockSpec((1, 1, OUT), lambda s, t: (_e_of(s), 0, 0))
    return pl.pallas_call(
        _experts_body,
        grid=(E, NT),
        in_specs=[
            pl.BlockSpec((N, D), lambda s, t: (0, 0)),
            wspec, bspec, wspec, bspec, wspec, bspec,
        ],
        out_specs=pl.BlockSpec((1, BT, OUT), lambda s, t: (_e_of(s), t, 0)),
        out_shape=jax.ShapeDtypeStruct((E, N, OUT), jnp.float32),
        compiler_params=pltpu.CompilerParams(
            dimension_semantics=("arbitrary", "arbitrary")),
    )(xb, W1b, b1.reshape(E, 1, OUT), W2b, b2.reshape(E, 1, OUT),
      W3b, b3.reshape(E, 1, OUT))


def _combine_body(eo_ref, gates_ref, idx_ref, out_ref):
    g1 = gates_ref[:, 0:1]
    g2 = gates_ref[:, 1:2]
    i1 = idx_ref[:, 0:1]
    i2 = idx_ref[:, 1:2]
    acc = jnp.zeros((BT, OUT), jnp.float32)
    for e in range(E):
        w = jnp.where(i1 == e, g1, 0.0) + jnp.where(i2 == e, g2, 0.0)
        acc = acc + w * eo_ref[e]
    out_ref[...] = acc


def _combine(eo, gates, idx):
    return pl.pallas_call(
        _combine_body,
        grid=(NT,),
        in_specs=[
            pl.BlockSpec((E, BT, OUT), lambda t: (0, t, 0)),
            pl.BlockSpec((BT, K), lambda t: (t, 0)),
            pl.BlockSpec((BT, K), lambda t: (t, 0)),
        ],
        out_specs=pl.BlockSpec((BT, OUT), lambda t: (t, 0)),
        out_shape=jax.ShapeDtypeStruct((N, OUT), jnp.float32),
        compiler_params=pltpu.CompilerParams(
            dimension_semantics=("arbitrary",)),
    )(eo, gates, idx)


def kernel(x, edge_attr, w_gating, W_ep, b_ep, W1, b1, W2, b2, W3, b3):
    xb = x.astype(jnp.bfloat16)
    gates, idx, loss = _gating(xb, edge_attr, w_gating, W_ep, b_ep)
    eo = _experts(xb, W1.astype(jnp.bfloat16), b1,
                  W2.astype(jnp.bfloat16), b2,
                  W3.astype(jnp.bfloat16), b3)
    final = _combine(eo, gates, idx)
    return final, eo, loss[0, 0]
